# baseline (speedup 1.0000x reference)
import jax
import jax.numpy as jnp
from jax.experimental import pallas as pl
from jax.experimental.pallas import tpu as pltpu

_K = 16
_KP1 = 17
_N = 2048
_D = 64
_R = 1024


def _edge_kernel(xt_ref, xq_ref, xs_ref, sqc_ref, sqr_ref, edge_ref, idx_ref):
    xt = xt_ref[0]
    xq = xq_ref[0]
    mm = jax.lax.dot_general(
        xt, xq, (((1,), (0,)), ((), ())),
        preferred_element_type=jnp.float32)
    d2 = (sqc_ref[0] + sqr_ref[0]) - 2.0 * mm
    dist = jnp.sqrt(jnp.maximum(d2, 0.0))
    iota = jax.lax.broadcasted_iota(jnp.int32, (_N, _R), 0)
    iota8 = jax.lax.broadcasted_iota(jnp.int32, (8, _R), 0)
    xs = xs_ref[0]
    edge_ref[0, 0:_D, :, :] = jnp.broadcast_to(xq[:, None, :], (_D, _K, _R))
    dT = dist
    nch = _N // 8
    nchain = 4
    cpc = nch // nchain
    for t in range(_KP1):
        vs, cs = [], []
        for ch in range(nchain):
            b0 = ch * cpc
            vacc = dT[b0 * 8:(b0 + 1) * 8, :]
            cacc = jnp.full((8, _R), b0, jnp.int32)
            for c in range(b0 + 1, b0 + cpc):
                v = dT[c * 8:(c + 1) * 8, :]
                take = v < vacc
                vacc = jnp.where(take, v, vacc)
                cacc = jnp.where(take, c, cacc)
            vs.append(vacc)
            cs.append(cacc)
        while len(vs) > 1:
            take = vs[1] < vs[0]
            vs = [jnp.where(take, vs[1], vs[0])] + vs[2:]
            cs = [jnp.where(take, cs[1], cs[0])] + cs[2:]
        fidx = cs[0] * 8 + iota8
        vcur, icur = vs[0], fidx
        for sh in (4, 2, 1):
            vb = pltpu.roll(vcur, 8 - sh, axis=0)
            ib = pltpu.bitcast(
                pltpu.roll(pltpu.bitcast(icur, jnp.float32), 8 - sh, axis=0),
                jnp.int32)
            take = (vb < vcur) | ((vb == vcur) & (ib < icur))
            vcur = jnp.where(take, vb, vcur)
            icur = jnp.where(take, ib, icur)
        j = icur[0:1, :]
        hit = iota == j
        if t >= 1:
            oh = hit.astype(jnp.bfloat16)
            nb2 = jax.lax.dot_general(
                xs, oh, (((1,), (0,)), ((), ())),
                preferred_element_type=jnp.float32)
            nb = nb2[0:_D] + nb2[_D:2 * _D]
            edge_ref[0, _D:2 * _D, t - 1, :] = nb - xq
            idx_ref[0, t - 1:t, :] = j
        if t < _KP1 - 1:
            dT = jnp.where(hit, jnp.float32(jnp.inf), dT)


def _build_call(B):
    return pl.pallas_call(
        _edge_kernel,
        grid=(B, _N // _R),
        in_specs=[
            pl.BlockSpec((1, _N, _D), lambda b, r: (b, 0, 0)),
            pl.BlockSpec((1, _D, _R), lambda b, r: (b, 0, r)),
            pl.BlockSpec((1, 2 * _D, _N), lambda b, r: (b, 0, 0)),
            pl.BlockSpec((1, _N, 1), lambda b, r: (b, 0, 0)),
            pl.BlockSpec((1, 1, _R), lambda b, r: (b, 0, r)),
        ],
        out_specs=[
            pl.BlockSpec((1, 2 * _D, _K, _R), lambda b, r: (b, 0, 0, r)),
            pl.BlockSpec((1, _K, _R), lambda b, r: (b, 0, r)),
        ],
        out_shape=[
            jax.ShapeDtypeStruct((B, 2 * _D, _K, _N), jnp.float32),
            jax.ShapeDtypeStruct((B, _K, _N), jnp.int32),
        ],
    )


def kernel(point_cloud):
    B, D, N = point_cloud.shape
    xt = jnp.transpose(point_cloud, (0, 2, 1))
    sq = jnp.sum(xt * xt, axis=-1)
    hi = point_cloud.astype(jnp.bfloat16)
    lo = (point_cloud - hi.astype(jnp.float32)).astype(jnp.bfloat16)
    xs = jnp.concatenate([hi, lo], axis=1)
    sqc = sq[:, :, None]
    sqr = sq[:, None, :]
    edge, idx = _build_call(B)(xt, point_cloud, xs, sqc, sqr)
    edge_feature = jnp.transpose(edge, (0, 1, 3, 2))
    idx_out = jnp.transpose(idx, (0, 2, 1)).reshape(B, N * _K)
    return (edge_feature, idx_out)


# speedup vs baseline: 1.0484x; 1.0484x over previous
import jax
import jax.numpy as jnp
from jax.experimental import pallas as pl
from jax.experimental.pallas import tpu as pltpu

_K = 16
_KP1 = 17
_N = 2048
_D = 64
_R = 512


def _edge_kernel(xt_ref, xq_ref, xs_ref, sqc_ref, sqr_ref, edge_ref, idx_ref):
    xt = xt_ref[0]
    xq = xq_ref[0]
    mm = jax.lax.dot_general(
        xt, xq, (((1,), (0,)), ((), ())),
        preferred_element_type=jnp.float32)
    d2 = (sqc_ref[0] + sqr_ref[0]) - 2.0 * mm
    dist = jnp.sqrt(jnp.maximum(d2, 0.0))
    iota = jax.lax.broadcasted_iota(jnp.int32, (_N, _R), 0)
    iota8 = jax.lax.broadcasted_iota(jnp.int32, (8, _R), 0)
    xs = xs_ref[0]
    edge_ref[0, 0:_D, :, :] = jnp.broadcast_to(xq[:, None, :], (_D, _K, _R))
    dT = dist
    nch = _N // 8
    nchain = 4
    cpc = nch // nchain
    for t in range(_KP1):
        vs, cs = [], []
        for ch in range(nchain):
            b0 = ch * cpc
            vacc = dT[b0 * 8:(b0 + 1) * 8, :]
            cacc = jnp.full((8, _R), b0, jnp.int32)
            for c in range(b0 + 1, b0 + cpc):
                v = dT[c * 8:(c + 1) * 8, :]
                take = v < vacc
                vacc = jnp.where(take, v, vacc)
                cacc = jnp.where(take, c, cacc)
            vs.append(vacc)
            cs.append(cacc)
        while len(vs) > 1:
            take = vs[1] < vs[0]
            vs = [jnp.where(take, vs[1], vs[0])] + vs[2:]
            cs = [jnp.where(take, cs[1], cs[0])] + cs[2:]
        fidx = cs[0] * 8 + iota8
        vcur, icur = vs[0], fidx
        for sh in (4, 2, 1):
            vb = pltpu.roll(vcur, 8 - sh, axis=0)
            ib = pltpu.bitcast(
                pltpu.roll(pltpu.bitcast(icur, jnp.float32), 8 - sh, axis=0),
                jnp.int32)
            take = (vb < vcur) | ((vb == vcur) & (ib < icur))
            vcur = jnp.where(take, vb, vcur)
            icur = jnp.where(take, ib, icur)
        j = icur[0:1, :]
        hit = iota == j
        if t >= 1:
            oh = hit.astype(jnp.bfloat16)
            nb2 = jax.lax.dot_general(
                xs, oh, (((1,), (0,)), ((), ())),
                preferred_element_type=jnp.float32)
            nb = nb2[0:_D] + nb2[_D:2 * _D]
            edge_ref[0, _D:2 * _D, t - 1, :] = nb - xq
            idx_ref[0, t - 1:t, :] = j
        if t < _KP1 - 1:
            dT = jnp.where(hit, jnp.float32(jnp.inf), dT)


def _build_call(B):
    return pl.pallas_call(
        _edge_kernel,
        grid=(B, _N // _R),
        in_specs=[
            pl.BlockSpec((1, _N, _D), lambda b, r: (b, 0, 0)),
            pl.BlockSpec((1, _D, _R), lambda b, r: (b, 0, r)),
            pl.BlockSpec((1, 2 * _D, _N), lambda b, r: (b, 0, 0)),
            pl.BlockSpec((1, _N, 1), lambda b, r: (b, 0, 0)),
            pl.BlockSpec((1, 1, _R), lambda b, r: (b, 0, r)),
        ],
        out_specs=[
            pl.BlockSpec((1, 2 * _D, _K, _R), lambda b, r: (b, 0, 0, r)),
            pl.BlockSpec((1, _K, _R), lambda b, r: (b, 0, r)),
        ],
        out_shape=[
            jax.ShapeDtypeStruct((B, 2 * _D, _K, _N), jnp.float32),
            jax.ShapeDtypeStruct((B, _K, _N), jnp.int32),
        ],
    )


def kernel(point_cloud):
    B, D, N = point_cloud.shape
    xt = jnp.transpose(point_cloud, (0, 2, 1))
    sq = jnp.sum(xt * xt, axis=-1)
    hi = point_cloud.astype(jnp.bfloat16)
    lo = (point_cloud - hi.astype(jnp.float32)).astype(jnp.bfloat16)
    xs = jnp.concatenate([hi, lo], axis=1)
    sqc = sq[:, :, None]
    sqr = sq[:, None, :]
    edge, idx = _build_call(B)(xt, point_cloud, xs, sqc, sqr)
    edge_feature = jnp.transpose(edge, (0, 1, 3, 2))
    idx_out = jnp.transpose(idx, (0, 2, 1)).reshape(B, N * _K)
    return (edge_feature, idx_out)
